# Initial kernel scaffold; baseline (speedup 1.0000x reference)
import jax
import jax.numpy as jnp
from jax.experimental import pallas as pl


def kernel(x_scalar, x_spherical, rbf, fcut, rsh, edge_index, W1, b1, W2, b2, Wr, br):
    raise NotImplementedError("write your pallas kernel here")


# speedup vs baseline: 1.2006x; 1.2006x over previous
import functools

import jax
import jax.numpy as jnp
import numpy as np
from jax.experimental import pallas as pl

N = 10000
E = 160000
NODE = 128
NUM_IRR = 224
SPH = 480
NB = 20
HID = NODE + NUM_IRR * 2

_reps = np.array([1] * 128 + [3] * 64 + [5] * 32)
_col_of = np.repeat(np.arange(NUM_IRR), _reps)
_SEL_TAIL = np.zeros((96, 352), dtype=np.float32)
for _j, _c in enumerate(_col_of[128:]):
    _SEL_TAIL[_c - 128, _j] = 1.0

BN = 1000
BE = 640


def _mlp_body(x_ref, w1_ref, b1_ref, w2_ref, b2_ref, o_ref):
    x = x_ref[...]
    h = jnp.dot(x, w1_ref[...], preferred_element_type=jnp.float32,
                precision=jax.lax.Precision.HIGHEST) + b1_ref[...]
    h = h * jax.nn.sigmoid(h)
    o_ref[...] = jnp.dot(h, w2_ref[...], preferred_element_type=jnp.float32,
                         precision=jax.lax.Precision.HIGHEST) + b2_ref[...]


def _edge_body(gso_ref, gsph_ref, rbf_ref, fcut_ref, rsh_ref,
               wr_ref, br_ref, sel_ref, ms_ref, msph_ref):
    fw = jnp.dot(rbf_ref[...], wr_ref[...], preferred_element_type=jnp.float32,
                 precision=jax.lax.Precision.HIGHEST) + br_ref[...]
    fw = fw * fcut_ref[...]
    fo = gso_ref[...] * fw
    gs = fo[:, :NUM_IRR]
    ge = fo[:, NUM_IRR:2 * NUM_IRR]
    ms_ref[...] = fo[:, 2 * NUM_IRR:]
    sel = sel_ref[...]

    def rep(g):
        tail = jnp.dot(g[:, 128:].astype(jnp.bfloat16), sel.astype(jnp.bfloat16),
                       preferred_element_type=jnp.float32)
        return jnp.concatenate([g[:, :128], tail], axis=1)

    msph_ref[...] = gsph_ref[...] * rep(gs) + rsh_ref[...] * rep(ge)


def _mlp(x_scalar, W1, b1, W2, b2):
    return pl.pallas_call(
        _mlp_body,
        grid=(N // BN,),
        in_specs=[
            pl.BlockSpec((BN, NODE), lambda i: (i, 0)),
            pl.BlockSpec((NODE, NODE), lambda i: (0, 0)),
            pl.BlockSpec((NODE,), lambda i: (0,)),
            pl.BlockSpec((NODE, HID), lambda i: (0, 0)),
            pl.BlockSpec((HID,), lambda i: (0,)),
        ],
        out_specs=pl.BlockSpec((BN, HID), lambda i: (i, 0)),
        out_shape=jax.ShapeDtypeStruct((N, HID), jnp.float32),
    )(x_scalar, W1, b1, W2, b2)


def _edge_math(g_so, g_sph, rbf, fcut, rsh, Wr, br, sel):
    return pl.pallas_call(
        _edge_body,
        grid=(E // BE,),
        in_specs=[
            pl.BlockSpec((BE, HID), lambda i: (i, 0)),
            pl.BlockSpec((BE, SPH), lambda i: (i, 0)),
            pl.BlockSpec((BE, NB), lambda i: (i, 0)),
            pl.BlockSpec((BE, 1), lambda i: (i, 0)),
            pl.BlockSpec((BE, SPH), lambda i: (i, 0)),
            pl.BlockSpec((NB, HID), lambda i: (0, 0)),
            pl.BlockSpec((HID,), lambda i: (0,)),
            pl.BlockSpec((96, 352), lambda i: (0, 0)),
        ],
        out_specs=[
            pl.BlockSpec((BE, NODE), lambda i: (i, 0)),
            pl.BlockSpec((BE, SPH), lambda i: (i, 0)),
        ],
        out_shape=[
            jax.ShapeDtypeStruct((E, NODE), jnp.float32),
            jax.ShapeDtypeStruct((E, SPH), jnp.float32),
        ],
    )(g_so, g_sph, rbf, fcut, rsh, Wr, br, sel)


def kernel(x_scalar, x_spherical, rbf, fcut, rsh, edge_index, W1, b1, W2, b2, Wr, br):
    sel = jnp.asarray(_SEL_TAIL)
    scalar_out = _mlp(x_scalar, W1, b1, W2, b2)
    dst = edge_index[1]
    src = edge_index[0]
    g_so = jnp.take(scalar_out, dst, axis=0)
    g_sph = jnp.take(x_spherical, dst, axis=0)
    msg_scalar, msg_sph = _edge_math(g_so, g_sph, rbf, fcut, rsh, Wr, br, sel)
    new_scalar = x_scalar + jax.ops.segment_sum(msg_scalar, src, num_segments=N)
    new_spherical = x_spherical + jax.ops.segment_sum(msg_sph, src, num_segments=N)
    return (new_scalar, new_spherical)


# speedup vs baseline: 1.6175x; 1.3472x over previous
import functools

import jax
import jax.numpy as jnp
import numpy as np
from jax import lax
from jax.experimental import pallas as pl
from jax.experimental.pallas import tpu as pltpu
from jax.experimental.pallas import tpu_sc as plsc

N = 10000
E = 160000
NODE = 128
NUM_IRR = 224
SPH = 480
NB = 20
HID = NODE + NUM_IRR * 2

_reps = np.array([1] * 128 + [3] * 64 + [5] * 32)
_col_of = np.repeat(np.arange(NUM_IRR), _reps)
_SEL_TAIL = np.zeros((96, 352), dtype=np.float32)
for _j, _c in enumerate(_col_of[128:]):
    _SEL_TAIL[_c - 128, _j] = 1.0

BN = 1000
BE = 640
HIDP = 640
SPHP = 512


def _mlp_body(x_ref, w1_ref, b1_ref, w2_ref, b2_ref, o_ref):
    x = x_ref[...]
    h = jnp.dot(x, w1_ref[...], preferred_element_type=jnp.float32,
                precision=jax.lax.Precision.HIGHEST) + b1_ref[...]
    h = h * jax.nn.sigmoid(h)
    o_ref[...] = jnp.dot(h, w2_ref[...], preferred_element_type=jnp.float32,
                         precision=jax.lax.Precision.HIGHEST) + b2_ref[...]


def _edge_body(gso_ref, gsph_ref, rbf_ref, fcut_ref, rsh_ref,
               wr_ref, br_ref, sel_ref, ms_ref, msph_ref):
    fw = jnp.dot(rbf_ref[...], wr_ref[...], preferred_element_type=jnp.float32,
                 precision=jax.lax.Precision.HIGHEST) + br_ref[...]
    fw = fw * fcut_ref[...]
    fo = gso_ref[...] * fw
    gs = fo[:, :NUM_IRR]
    ge = fo[:, NUM_IRR:2 * NUM_IRR]
    ms_ref[...] = fo[:, 2 * NUM_IRR:HID]
    sel = sel_ref[...]

    def rep(g):
        tail = jnp.dot(g[:, 128:].astype(jnp.bfloat16), sel.astype(jnp.bfloat16),
                       preferred_element_type=jnp.float32)
        return jnp.concatenate([g[:, :128], tail], axis=1)

    msph_ref[...] = gsph_ref[:, :SPH] * rep(gs) + rsh_ref[...] * rep(ge)


def _mlp(x_scalar, W1, b1, W2, b2):
    return pl.pallas_call(
        _mlp_body,
        grid=(N // BN,),
        in_specs=[
            pl.BlockSpec((BN, NODE), lambda i: (i, 0)),
            pl.BlockSpec((NODE, NODE), lambda i: (0, 0)),
            pl.BlockSpec((NODE,), lambda i: (0,)),
            pl.BlockSpec((NODE, HIDP), lambda i: (0, 0)),
            pl.BlockSpec((HIDP,), lambda i: (0,)),
        ],
        out_specs=pl.BlockSpec((BN, HIDP), lambda i: (i, 0)),
        out_shape=jax.ShapeDtypeStruct((N, HIDP), jnp.float32),
    )(x_scalar, W1, b1, W2, b2)


def _edge_math(g_so, g_sph, rbf, fcut, rsh, Wr, br, sel):
    return pl.pallas_call(
        _edge_body,
        grid=(E // BE,),
        in_specs=[
            pl.BlockSpec((BE, HIDP), lambda i: (i, 0)),
            pl.BlockSpec((BE, SPHP), lambda i: (i, 0)),
            pl.BlockSpec((BE, NB), lambda i: (i, 0)),
            pl.BlockSpec((BE, 1), lambda i: (i, 0)),
            pl.BlockSpec((BE, SPH), lambda i: (i, 0)),
            pl.BlockSpec((NB, HIDP), lambda i: (0, 0)),
            pl.BlockSpec((HIDP,), lambda i: (0,)),
            pl.BlockSpec((96, 352), lambda i: (0, 0)),
        ],
        out_specs=[
            pl.BlockSpec((BE, NODE), lambda i: (i, 0)),
            pl.BlockSpec((BE, SPH), lambda i: (i, 0)),
        ],
        out_shape=[
            jax.ShapeDtypeStruct((E, NODE), jnp.float32),
            jax.ShapeDtypeStruct((E, SPH), jnp.float32),
        ],
    )(g_so, g_sph, rbf, fcut, rsh, Wr, br, sel)


_NC, _NS = 2, 16
_NW = _NC * _NS
_EPW = E // _NW
_GCH = 40

_SC_MESH = plsc.VectorSubcoreMesh(core_axis_name="c", subcore_axis_name="s")


def _gather_body(so_hbm, sph_hbm, dst_hbm, out_so, out_sph,
                 idx_v, so_v, sph_v, sem):
    wid = lax.axis_index("s") * _NC + lax.axis_index("c")
    base = wid * _EPW

    def step(j, carry):
        cb = base + j * _GCH
        pltpu.sync_copy(dst_hbm.at[pl.ds(cb, _GCH)], idx_v)
        a = pltpu.async_copy(so_hbm.at[idx_v], so_v, sem)
        b = pltpu.async_copy(sph_hbm.at[idx_v], sph_v, sem)
        a.wait()
        b.wait()
        pltpu.sync_copy(so_v, out_so.at[pl.ds(cb, _GCH)])
        pltpu.sync_copy(sph_v, out_sph.at[pl.ds(cb, _GCH)])
        return carry

    lax.fori_loop(0, _EPW // _GCH, step, 0)


def _sc_gather(scalar_out, x_spherical, dst):
    return pl.kernel(
        _gather_body,
        out_type=[
            jax.ShapeDtypeStruct((E, HIDP), jnp.float32),
            jax.ShapeDtypeStruct((E, SPHP), jnp.float32),
        ],
        mesh=_SC_MESH,
        scratch_types=[
            pltpu.VMEM((_GCH,), jnp.int32),
            pltpu.VMEM((_GCH, HIDP), jnp.float32),
            pltpu.VMEM((_GCH, SPHP), jnp.float32),
            pltpu.SemaphoreType.DMA,
        ],
    )(scalar_out, x_spherical, dst)


def kernel(x_scalar, x_spherical, rbf, fcut, rsh, edge_index, W1, b1, W2, b2, Wr, br):
    sel = jnp.asarray(_SEL_TAIL)
    W2p = jnp.pad(W2, ((0, 0), (0, HIDP - HID)))
    b2p = jnp.pad(b2, (0, HIDP - HID))
    Wrp = jnp.pad(Wr, ((0, 0), (0, HIDP - HID)))
    brp = jnp.pad(br, (0, HIDP - HID))
    x_sph_p = jnp.pad(x_spherical, ((0, 0), (0, SPHP - SPH)))
    scalar_out = _mlp(x_scalar, W1, b1, W2p, b2p)
    dst = edge_index[1]
    src = edge_index[0]
    g_so, g_sph = _sc_gather(scalar_out, x_sph_p, dst)
    msg_scalar, msg_sph = _edge_math(g_so, g_sph, rbf, fcut, rsh, Wrp, brp, sel)
    new_scalar = x_scalar + jax.ops.segment_sum(msg_scalar, src, num_segments=N)
    new_spherical = x_spherical + jax.ops.segment_sum(msg_sph, src, num_segments=N)
    return (new_scalar, new_spherical)


# speedup vs baseline: 2.5750x; 1.5920x over previous
import functools

import jax
import jax.numpy as jnp
import numpy as np
from jax import lax
from jax.experimental import pallas as pl
from jax.experimental.pallas import tpu as pltpu
from jax.experimental.pallas import tpu_sc as plsc

N = 10000
E = 160000
NODE = 128
NUM_IRR = 224
SPH = 480
NB = 20
HID = NODE + NUM_IRR * 2

_reps = np.array([1] * 128 + [3] * 64 + [5] * 32)
_col_of = np.repeat(np.arange(NUM_IRR), _reps)
_SEL_TAIL = np.zeros((96, 352), dtype=np.float32)
for _j, _c in enumerate(_col_of[128:]):
    _SEL_TAIL[_c - 128, _j] = 1.0

BN = 1000
BE = 640
HIDP = 640
SPHP = 512


def _mlp_body(x_ref, w1_ref, b1_ref, w2_ref, b2_ref, o_ref):
    x = x_ref[...]
    h = jnp.dot(x, w1_ref[...], preferred_element_type=jnp.float32,
                precision=jax.lax.Precision.HIGHEST) + b1_ref[...]
    h = h * jax.nn.sigmoid(h)
    o_ref[...] = jnp.dot(h, w2_ref[...], preferred_element_type=jnp.float32,
                         precision=jax.lax.Precision.HIGHEST) + b2_ref[...]


def _edge_body(gso_ref, gsph_ref, rbf_ref, fcut_ref, rsh_ref,
               wr_ref, br_ref, sel_ref, msg_ref):
    fw = jnp.dot(rbf_ref[...], wr_ref[...], preferred_element_type=jnp.float32,
                 precision=jax.lax.Precision.HIGHEST) + br_ref[...]
    fw = fw * fcut_ref[...]
    fo = gso_ref[...] * fw
    gs = fo[:, :NUM_IRR]
    ge = fo[:, NUM_IRR:2 * NUM_IRR]
    ms = fo[:, 2 * NUM_IRR:HID]
    sel = sel_ref[...]

    def rep(g):
        tail = jnp.dot(g[:, 128:].astype(jnp.bfloat16), sel.astype(jnp.bfloat16),
                       preferred_element_type=jnp.float32)
        return jnp.concatenate([g[:, :128], tail], axis=1)

    msph = gsph_ref[:, :SPH] * rep(gs) + rsh_ref[...] * rep(ge)
    pad = jnp.zeros((msph.shape[0], HIDP - NODE - SPH), jnp.float32)
    msg_ref[...] = jnp.concatenate([ms, msph, pad], axis=1)


def _mlp(x_scalar, W1, b1, W2, b2):
    return pl.pallas_call(
        _mlp_body,
        grid=(N // BN,),
        in_specs=[
            pl.BlockSpec((BN, NODE), lambda i: (i, 0)),
            pl.BlockSpec((NODE, NODE), lambda i: (0, 0)),
            pl.BlockSpec((NODE,), lambda i: (0,)),
            pl.BlockSpec((NODE, HIDP), lambda i: (0, 0)),
            pl.BlockSpec((HIDP,), lambda i: (0,)),
        ],
        out_specs=pl.BlockSpec((BN, HIDP), lambda i: (i, 0)),
        out_shape=jax.ShapeDtypeStruct((N, HIDP), jnp.float32),
    )(x_scalar, W1, b1, W2, b2)


def _edge_math(g_so, g_sph, rbf, fcut, rsh, Wr, br, sel):
    return pl.pallas_call(
        _edge_body,
        grid=(E // BE,),
        in_specs=[
            pl.BlockSpec((BE, HIDP), lambda i: (i, 0)),
            pl.BlockSpec((BE, SPHP), lambda i: (i, 0)),
            pl.BlockSpec((BE, NB), lambda i: (i, 0)),
            pl.BlockSpec((BE, 1), lambda i: (i, 0)),
            pl.BlockSpec((BE, SPH), lambda i: (i, 0)),
            pl.BlockSpec((NB, HIDP), lambda i: (0, 0)),
            pl.BlockSpec((HIDP,), lambda i: (0,)),
            pl.BlockSpec((96, 352), lambda i: (0, 0)),
        ],
        out_specs=pl.BlockSpec((BE, HIDP), lambda i: (i, 0)),
        out_shape=jax.ShapeDtypeStruct((E, HIDP), jnp.float32),
    )(g_so, g_sph, rbf, fcut, rsh, Wr, br, sel)


_NC, _NS = 2, 16
_NW = _NC * _NS
_EPW = E // _NW
_GCH = 40

_SC_MESH = plsc.VectorSubcoreMesh(core_axis_name="c", subcore_axis_name="s")


def _gather_body(so_hbm, sph_hbm, dst_hbm, out_so, out_sph,
                 idx_v, so_v, sph_v, sem):
    wid = lax.axis_index("s") * _NC + lax.axis_index("c")
    base = wid * _EPW

    def step(j, carry):
        cb = base + j * _GCH
        pltpu.sync_copy(dst_hbm.at[pl.ds(cb, _GCH)], idx_v)
        a = pltpu.async_copy(so_hbm.at[idx_v], so_v, sem)
        b = pltpu.async_copy(sph_hbm.at[idx_v], sph_v, sem)
        a.wait()
        b.wait()
        pltpu.sync_copy(so_v, out_so.at[pl.ds(cb, _GCH)])
        pltpu.sync_copy(sph_v, out_sph.at[pl.ds(cb, _GCH)])
        return carry

    lax.fori_loop(0, _EPW // _GCH, step, 0)


def _sc_gather(scalar_out, x_spherical, dst):
    return pl.kernel(
        _gather_body,
        out_type=[
            jax.ShapeDtypeStruct((E, HIDP), jnp.float32),
            jax.ShapeDtypeStruct((E, SPHP), jnp.float32),
        ],
        mesh=_SC_MESH,
        scratch_types=[
            pltpu.VMEM((_GCH,), jnp.int32),
            pltpu.VMEM((_GCH, HIDP), jnp.float32),
            pltpu.VMEM((_GCH, SPHP), jnp.float32),
            pltpu.SemaphoreType.DMA,
        ],
    )(scalar_out, x_spherical, dst)


_SCH = 128
_NFULL = _EPW // _SCH
_REM = _EPW - _NFULL * _SCH
_NPASS = HIDP // 128
_EPC = E // _NC


def _scatter_body(msg_hbm, src_hbm, zeros_hbm, out_hbm,
                  idx_a, idx_b, idx_r, msg_a, msg_b, msg_r,
                  acc_sh, sem_a, sem_b):
    c = lax.axis_index("c")
    s = lax.axis_index("s")
    ebase = c * _EPC + s * _EPW

    def start(chunk, idx_v, msg_v, sem, col):
        eb = pl.multiple_of(ebase + chunk * _SCH, 8)
        pltpu.async_copy(src_hbm.at[pl.ds(eb, _SCH)], idx_v, sem)
        pltpu.async_copy(msg_hbm.at[pl.ds(eb, _SCH), pl.ds(col, 128)],
                         msg_v, sem)

    def wait_and_scatter(idx_v, msg_v, sem, col):
        pltpu.make_async_copy(src_hbm.at[pl.ds(0, _SCH)], idx_v, sem).wait()
        pltpu.make_async_copy(msg_hbm.at[pl.ds(0, _SCH), pl.ds(col, 128)],
                              msg_v, sem).wait()
        pltpu.sync_copy(msg_v, acc_sh.at[idx_v], add=True)

    def one_pass(p, carry):
        col = pl.multiple_of(p * 128, 128)
        @pl.when(s < _NS - 1)
        def _():
            pltpu.sync_copy(zeros_hbm.at[pl.ds(0, 624)],
                            acc_sh.at[pl.ds(s * 624, 624)])

        @pl.when(s == _NS - 1)
        def _():
            pltpu.sync_copy(zeros_hbm.at[pl.ds(0, 640)],
                            acc_sh.at[pl.ds(9360, 640)])

        plsc.subcore_barrier()

        start(0, idx_a, msg_a, sem_a, col)

        def pair(j, carry):
            @pl.when(2 * j + 1 < _NFULL)
            def _():
                start(2 * j + 1, idx_b, msg_b, sem_b, col)

            wait_and_scatter(idx_a, msg_a, sem_a, col)

            @pl.when(2 * j + 2 < _NFULL)
            def _():
                start(2 * j + 2, idx_a, msg_a, sem_a, col)

            @pl.when(2 * j + 1 < _NFULL)
            def _():
                wait_and_scatter(idx_b, msg_b, sem_b, col)

            return carry

        lax.fori_loop(0, (_NFULL + 1) // 2, pair, 0)

        rb = pl.multiple_of(ebase + _NFULL * _SCH, 8)
        pltpu.sync_copy(src_hbm.at[pl.ds(rb, _REM)], idx_r)
        pltpu.sync_copy(msg_hbm.at[pl.ds(rb, _REM), pl.ds(col, 128)], msg_r)
        pltpu.sync_copy(msg_r, acc_sh.at[idx_r], add=True)
        plsc.subcore_barrier()

        @pl.when(s < _NS - 1)
        def _():
            pltpu.sync_copy(acc_sh.at[pl.ds(s * 624, 624)],
                            out_hbm.at[p, c].at[pl.ds(s * 624, 624)])

        @pl.when(s == _NS - 1)
        def _():
            pltpu.sync_copy(acc_sh.at[pl.ds(9360, 640)],
                            out_hbm.at[p, c].at[pl.ds(9360, 640)])

        plsc.subcore_barrier()
        return carry

    lax.fori_loop(0, _NPASS, one_pass, 0)


def _sc_scatter(msg, src, zeros):
    return pl.kernel(
        _scatter_body,
        out_type=jax.ShapeDtypeStruct((_NPASS, _NC, N, 128), jnp.float32),
        mesh=_SC_MESH,
        scratch_types=[
            pltpu.VMEM((_SCH,), jnp.int32),
            pltpu.VMEM((_SCH,), jnp.int32),
            pltpu.VMEM((_REM,), jnp.int32),
            pltpu.VMEM((_SCH, 128), jnp.float32),
            pltpu.VMEM((_SCH, 128), jnp.float32),
            pltpu.VMEM((_REM, 128), jnp.float32),
            pltpu.VMEM_SHARED((N, 128), jnp.float32),
            pltpu.SemaphoreType.DMA,
            pltpu.SemaphoreType.DMA,
        ],
    )(msg, src, zeros)


def kernel(x_scalar, x_spherical, rbf, fcut, rsh, edge_index, W1, b1, W2, b2, Wr, br):
    sel = jnp.asarray(_SEL_TAIL)
    W2p = jnp.pad(W2, ((0, 0), (0, HIDP - HID)))
    b2p = jnp.pad(b2, (0, HIDP - HID))
    Wrp = jnp.pad(Wr, ((0, 0), (0, HIDP - HID)))
    brp = jnp.pad(br, (0, HIDP - HID))
    x_sph_p = jnp.pad(x_spherical, ((0, 0), (0, SPHP - SPH)))
    scalar_out = _mlp(x_scalar, W1, b1, W2p, b2p)
    dst = edge_index[1]
    src = edge_index[0]
    g_so, g_sph = _sc_gather(scalar_out, x_sph_p, dst)
    msg = _edge_math(g_so, g_sph, rbf, fcut, rsh, Wrp, brp, sel)
    zeros = jnp.zeros((640, 128), jnp.float32)
    parts = _sc_scatter(msg, src, zeros)
    part = parts[:, 0] + parts[:, 1]
    new_scalar = x_scalar + part[0]
    sph_sum = jnp.concatenate([part[1], part[2], part[3], part[4]],
                              axis=1)[:, :SPH]
    new_spherical = x_spherical + sph_sum
    return (new_scalar, new_spherical)


# speedup vs baseline: 2.6096x; 1.0134x over previous
import functools

import jax
import jax.numpy as jnp
import numpy as np
from jax import lax
from jax.experimental import pallas as pl
from jax.experimental.pallas import tpu as pltpu
from jax.experimental.pallas import tpu_sc as plsc

N = 10000
E = 160000
NODE = 128
NUM_IRR = 224
SPH = 480
NB = 20
HID = NODE + NUM_IRR * 2

_reps = np.array([1] * 128 + [3] * 64 + [5] * 32)
_col_of = np.repeat(np.arange(NUM_IRR), _reps)
_SEL_TAIL = np.zeros((96, 352), dtype=np.float32)
for _j, _c in enumerate(_col_of[128:]):
    _SEL_TAIL[_c - 128, _j] = 1.0

BN = 1000
BE = 640
HIDP = 640
SPHP = 512


def _mlp_body(x_ref, w1_ref, b1_ref, w2_ref, b2_ref, o_ref):
    x = x_ref[...]
    h = jnp.dot(x, w1_ref[...], preferred_element_type=jnp.float32,
                precision=jax.lax.Precision.HIGHEST) + b1_ref[...]
    h = h * jax.nn.sigmoid(h)
    o_ref[...] = jnp.dot(h, w2_ref[...], preferred_element_type=jnp.float32,
                         precision=jax.lax.Precision.HIGHEST) + b2_ref[...]


def _edge_body(gso_ref, gsph_ref, rbf_ref, fcut_ref, rsh_ref,
               wr_ref, br_ref, sel_ref, msg_ref):
    fw = jnp.dot(rbf_ref[...], wr_ref[...], preferred_element_type=jnp.float32,
                 precision=jax.lax.Precision.HIGHEST) + br_ref[...]
    fw = fw * fcut_ref[...]
    fo = gso_ref[...] * fw
    gs = fo[:, :NUM_IRR]
    ge = fo[:, NUM_IRR:2 * NUM_IRR]
    ms = fo[:, 2 * NUM_IRR:HID]
    sel = sel_ref[...]

    def rep(g):
        tail = jnp.dot(g[:, 128:].astype(jnp.bfloat16), sel.astype(jnp.bfloat16),
                       preferred_element_type=jnp.float32)
        return jnp.concatenate([g[:, :128], tail], axis=1)

    msph = gsph_ref[:, :SPH] * rep(gs) + rsh_ref[...] * rep(ge)
    pad = jnp.zeros((msph.shape[0], HIDP - NODE - SPH), jnp.float32)
    msg_ref[...] = jnp.concatenate([ms, msph, pad], axis=1)


def _mlp(x_scalar, W1, b1, W2, b2):
    return pl.pallas_call(
        _mlp_body,
        grid=(N // BN,),
        in_specs=[
            pl.BlockSpec((BN, NODE), lambda i: (i, 0)),
            pl.BlockSpec((NODE, NODE), lambda i: (0, 0)),
            pl.BlockSpec((NODE,), lambda i: (0,)),
            pl.BlockSpec((NODE, HIDP), lambda i: (0, 0)),
            pl.BlockSpec((HIDP,), lambda i: (0,)),
        ],
        out_specs=pl.BlockSpec((BN, HIDP), lambda i: (i, 0)),
        out_shape=jax.ShapeDtypeStruct((N, HIDP), jnp.float32),
    )(x_scalar, W1, b1, W2, b2)


def _edge_math(g_so, g_sph, rbf, fcut, rsh, Wr, br, sel):
    return pl.pallas_call(
        _edge_body,
        grid=(E // BE,),
        in_specs=[
            pl.BlockSpec((BE, HIDP), lambda i: (i, 0)),
            pl.BlockSpec((BE, SPHP), lambda i: (i, 0)),
            pl.BlockSpec((BE, NB), lambda i: (i, 0)),
            pl.BlockSpec((BE, 1), lambda i: (i, 0)),
            pl.BlockSpec((BE, SPH), lambda i: (i, 0)),
            pl.BlockSpec((NB, HIDP), lambda i: (0, 0)),
            pl.BlockSpec((HIDP,), lambda i: (0,)),
            pl.BlockSpec((96, 352), lambda i: (0, 0)),
        ],
        out_specs=pl.BlockSpec((BE, HIDP), lambda i: (i, 0)),
        out_shape=jax.ShapeDtypeStruct((E, HIDP), jnp.float32),
    )(g_so, g_sph, rbf, fcut, rsh, Wr, br, sel)


_NC, _NS = 2, 16
_NW = _NC * _NS
_EPW = E // _NW
_GCH = 40

_SC_MESH = plsc.VectorSubcoreMesh(core_axis_name="c", subcore_axis_name="s")


_NGCH = _EPW // _GCH


def _gather_body(so_hbm, sph_hbm, dst_hbm, out_so, out_sph,
                 idx_all, so_a, so_b, sph_a, sph_b,
                 gsem_a, gsem_b, wsem_a, wsem_b):
    wid = lax.axis_index("s") * _NC + lax.axis_index("c")
    base = wid * _EPW
    pltpu.sync_copy(dst_hbm.at[pl.ds(base, _EPW)], idx_all)

    def gstart(chunk, so_v, sph_v, gsem):
        off = pl.multiple_of(chunk * _GCH, 8)
        idx = idx_all.at[pl.ds(off, _GCH)]
        pltpu.async_copy(so_hbm.at[idx], so_v, gsem)
        pltpu.async_copy(sph_hbm.at[idx], sph_v, gsem)

    def gwait(so_v, sph_v, gsem):
        pltpu.make_async_copy(so_hbm.at[pl.ds(0, _GCH)], so_v, gsem).wait()
        pltpu.make_async_copy(sph_hbm.at[pl.ds(0, _GCH)], sph_v, gsem).wait()

    def wstart(chunk, so_v, sph_v, wsem):
        cb = pl.multiple_of(base + chunk * _GCH, 8)
        pltpu.async_copy(so_v, out_so.at[pl.ds(cb, _GCH)], wsem)
        pltpu.async_copy(sph_v, out_sph.at[pl.ds(cb, _GCH)], wsem)

    def wwait(so_v, sph_v, wsem):
        pltpu.make_async_copy(so_v, out_so.at[pl.ds(0, _GCH)], wsem).wait()
        pltpu.make_async_copy(sph_v, out_sph.at[pl.ds(0, _GCH)], wsem).wait()

    gstart(0, so_a, sph_a, gsem_a)
    gstart(1, so_b, sph_b, gsem_b)

    def pair(j, carry):
        gwait(so_a, sph_a, gsem_a)
        wstart(2 * j, so_a, sph_a, wsem_a)

        @pl.when(2 * j + 1 < _NGCH)
        def _():
            gwait(so_b, sph_b, gsem_b)
            wstart(2 * j + 1, so_b, sph_b, wsem_b)

        @pl.when(2 * j + 2 < _NGCH)
        def _():
            wwait(so_a, sph_a, wsem_a)
            gstart(2 * j + 2, so_a, sph_a, gsem_a)

        @pl.when(2 * j + 3 < _NGCH)
        def _():
            wwait(so_b, sph_b, wsem_b)
            gstart(2 * j + 3, so_b, sph_b, gsem_b)

        return carry

    lax.fori_loop(0, (_NGCH + 1) // 2, pair, 0)
    wwait(so_a, sph_a, wsem_a)
    wwait(so_b, sph_b, wsem_b)


def _sc_gather(scalar_out, x_spherical, dst):
    return pl.kernel(
        _gather_body,
        out_type=[
            jax.ShapeDtypeStruct((E, HIDP), jnp.float32),
            jax.ShapeDtypeStruct((E, SPHP), jnp.float32),
        ],
        mesh=_SC_MESH,
        scratch_types=[
            pltpu.VMEM((_EPW,), jnp.int32),
            pltpu.VMEM((_GCH, HIDP), jnp.float32),
            pltpu.VMEM((_GCH, HIDP), jnp.float32),
            pltpu.VMEM((_GCH, SPHP), jnp.float32),
            pltpu.VMEM((_GCH, SPHP), jnp.float32),
            pltpu.SemaphoreType.DMA,
            pltpu.SemaphoreType.DMA,
            pltpu.SemaphoreType.DMA,
            pltpu.SemaphoreType.DMA,
        ],
    )(scalar_out, x_spherical, dst)


_SCH = 128
_NFULL = _EPW // _SCH
_REM = _EPW - _NFULL * _SCH
_NPASS = HIDP // 128
_EPC = E // _NC


def _scatter_body(msg_hbm, src_hbm, zeros_hbm, out_hbm,
                  idx_a, idx_b, idx_r, msg_a, msg_b, msg_r,
                  acc_sh, sem_a, sem_b):
    c = lax.axis_index("c")
    s = lax.axis_index("s")
    ebase = c * _EPC + s * _EPW

    def start(chunk, idx_v, msg_v, sem, col):
        eb = pl.multiple_of(ebase + chunk * _SCH, 8)
        pltpu.async_copy(src_hbm.at[pl.ds(eb, _SCH)], idx_v, sem)
        pltpu.async_copy(msg_hbm.at[pl.ds(eb, _SCH), pl.ds(col, 128)],
                         msg_v, sem)

    def wait_and_scatter(idx_v, msg_v, sem, col):
        pltpu.make_async_copy(src_hbm.at[pl.ds(0, _SCH)], idx_v, sem).wait()
        pltpu.make_async_copy(msg_hbm.at[pl.ds(0, _SCH), pl.ds(col, 128)],
                              msg_v, sem).wait()
        pltpu.sync_copy(msg_v, acc_sh.at[idx_v], add=True)

    def one_pass(p, carry):
        col = pl.multiple_of(p * 128, 128)
        @pl.when(s < _NS - 1)
        def _():
            pltpu.sync_copy(zeros_hbm.at[pl.ds(0, 624)],
                            acc_sh.at[pl.ds(s * 624, 624)])

        @pl.when(s == _NS - 1)
        def _():
            pltpu.sync_copy(zeros_hbm.at[pl.ds(0, 640)],
                            acc_sh.at[pl.ds(9360, 640)])

        plsc.subcore_barrier()

        start(0, idx_a, msg_a, sem_a, col)

        def pair(j, carry):
            @pl.when(2 * j + 1 < _NFULL)
            def _():
                start(2 * j + 1, idx_b, msg_b, sem_b, col)

            wait_and_scatter(idx_a, msg_a, sem_a, col)

            @pl.when(2 * j + 2 < _NFULL)
            def _():
                start(2 * j + 2, idx_a, msg_a, sem_a, col)

            @pl.when(2 * j + 1 < _NFULL)
            def _():
                wait_and_scatter(idx_b, msg_b, sem_b, col)

            return carry

        lax.fori_loop(0, (_NFULL + 1) // 2, pair, 0)

        rb = pl.multiple_of(ebase + _NFULL * _SCH, 8)
        pltpu.sync_copy(src_hbm.at[pl.ds(rb, _REM)], idx_r)
        pltpu.sync_copy(msg_hbm.at[pl.ds(rb, _REM), pl.ds(col, 128)], msg_r)
        pltpu.sync_copy(msg_r, acc_sh.at[idx_r], add=True)
        plsc.subcore_barrier()

        @pl.when(s < _NS - 1)
        def _():
            pltpu.sync_copy(acc_sh.at[pl.ds(s * 624, 624)],
                            out_hbm.at[p, c].at[pl.ds(s * 624, 624)])

        @pl.when(s == _NS - 1)
        def _():
            pltpu.sync_copy(acc_sh.at[pl.ds(9360, 640)],
                            out_hbm.at[p, c].at[pl.ds(9360, 640)])

        plsc.subcore_barrier()
        return carry

    lax.fori_loop(0, _NPASS, one_pass, 0)


def _sc_scatter(msg, src, zeros):
    return pl.kernel(
        _scatter_body,
        out_type=jax.ShapeDtypeStruct((_NPASS, _NC, N, 128), jnp.float32),
        mesh=_SC_MESH,
        scratch_types=[
            pltpu.VMEM((_SCH,), jnp.int32),
            pltpu.VMEM((_SCH,), jnp.int32),
            pltpu.VMEM((_REM,), jnp.int32),
            pltpu.VMEM((_SCH, 128), jnp.float32),
            pltpu.VMEM((_SCH, 128), jnp.float32),
            pltpu.VMEM((_REM, 128), jnp.float32),
            pltpu.VMEM_SHARED((N, 128), jnp.float32),
            pltpu.SemaphoreType.DMA,
            pltpu.SemaphoreType.DMA,
        ],
    )(msg, src, zeros)


def kernel(x_scalar, x_spherical, rbf, fcut, rsh, edge_index, W1, b1, W2, b2, Wr, br):
    sel = jnp.asarray(_SEL_TAIL)
    W2p = jnp.pad(W2, ((0, 0), (0, HIDP - HID)))
    b2p = jnp.pad(b2, (0, HIDP - HID))
    Wrp = jnp.pad(Wr, ((0, 0), (0, HIDP - HID)))
    brp = jnp.pad(br, (0, HIDP - HID))
    x_sph_p = jnp.pad(x_spherical, ((0, 0), (0, SPHP - SPH)))
    scalar_out = _mlp(x_scalar, W1, b1, W2p, b2p)
    dst = edge_index[1]
    src = edge_index[0]
    g_so, g_sph = _sc_gather(scalar_out, x_sph_p, dst)
    msg = _edge_math(g_so, g_sph, rbf, fcut, rsh, Wrp, brp, sel)
    zeros = jnp.zeros((640, 128), jnp.float32)
    parts = _sc_scatter(msg, src, zeros)
    part = parts[:, 0] + parts[:, 1]
    new_scalar = x_scalar + part[0]
    sph_sum = jnp.concatenate([part[1], part[2], part[3], part[4]],
                              axis=1)[:, :SPH]
    new_spherical = x_spherical + sph_sum
    return (new_scalar, new_spherical)


# speedup vs baseline: 2.7794x; 1.0650x over previous
import functools

import jax
import jax.numpy as jnp
import numpy as np
from jax import lax
from jax.experimental import pallas as pl
from jax.experimental.pallas import tpu as pltpu
from jax.experimental.pallas import tpu_sc as plsc

N = 10000
E = 160000
NODE = 128
NUM_IRR = 224
SPH = 480
NB = 20
HID = NODE + NUM_IRR * 2

_reps = np.array([1] * 128 + [3] * 64 + [5] * 32)
_col_of = np.repeat(np.arange(NUM_IRR), _reps)
_SEL_TAIL = np.zeros((96, 352), dtype=np.float32)
for _j, _c in enumerate(_col_of[128:]):
    _SEL_TAIL[_c - 128, _j] = 1.0

BN = 1000
BE = 640
HIDP = 640
SPHP = 512


def _mlp_body(x_ref, xsph_ref, w1_ref, b1_ref, w2_ref, b2_ref, o_ref):
    x = x_ref[...]
    h = jnp.dot(x, w1_ref[...], preferred_element_type=jnp.float32,
                precision=jax.lax.Precision.HIGHEST) + b1_ref[...]
    h = h * jax.nn.sigmoid(h)
    so = jnp.dot(h, w2_ref[...], preferred_element_type=jnp.float32,
                 precision=jax.lax.Precision.HIGHEST) + b2_ref[...]
    pad = jnp.zeros((so.shape[0], SPHP - SPH), jnp.float32)
    o_ref[...] = jnp.concatenate([so, xsph_ref[...], pad], axis=1)


def _edge_body(gtab_ref, rbf_ref, fcut_ref, rsh_ref,
               wr_ref, br_ref, sel_ref, msg_ref):
    fw = jnp.dot(rbf_ref[...], wr_ref[...], preferred_element_type=jnp.float32,
                 precision=jax.lax.Precision.HIGHEST) + br_ref[...]
    fw = fw * fcut_ref[...]
    fo = gtab_ref[:, :HIDP] * fw
    gs = fo[:, :NUM_IRR]
    ge = fo[:, NUM_IRR:2 * NUM_IRR]
    ms = fo[:, 2 * NUM_IRR:HID]
    sel = sel_ref[...]

    def rep(g):
        tail = jnp.dot(g[:, 128:].astype(jnp.bfloat16), sel.astype(jnp.bfloat16),
                       preferred_element_type=jnp.float32)
        return jnp.concatenate([g[:, :128], tail], axis=1)

    msph = gtab_ref[:, HIDP:HIDP + SPH] * rep(gs) + rsh_ref[...] * rep(ge)
    pad = jnp.zeros((msph.shape[0], HIDP - NODE - SPH), jnp.float32)
    msg_ref[...] = jnp.concatenate([ms, msph, pad], axis=1)


TBW = HIDP + SPHP


def _mlp(x_scalar, x_spherical, W1, b1, W2, b2):
    return pl.pallas_call(
        _mlp_body,
        grid=(N // BN,),
        in_specs=[
            pl.BlockSpec((BN, NODE), lambda i: (i, 0)),
            pl.BlockSpec((BN, SPH), lambda i: (i, 0)),
            pl.BlockSpec((NODE, NODE), lambda i: (0, 0)),
            pl.BlockSpec((NODE,), lambda i: (0,)),
            pl.BlockSpec((NODE, HIDP), lambda i: (0, 0)),
            pl.BlockSpec((HIDP,), lambda i: (0,)),
        ],
        out_specs=pl.BlockSpec((BN, TBW), lambda i: (i, 0)),
        out_shape=jax.ShapeDtypeStruct((N, TBW), jnp.float32),
    )(x_scalar, x_spherical, W1, b1, W2, b2)


def _edge_math(g_tab, rbf, fcut, rsh, Wr, br, sel):
    return pl.pallas_call(
        _edge_body,
        grid=(E // BE,),
        in_specs=[
            pl.BlockSpec((BE, TBW), lambda i: (i, 0)),
            pl.BlockSpec((BE, NB), lambda i: (i, 0)),
            pl.BlockSpec((BE, 1), lambda i: (i, 0)),
            pl.BlockSpec((BE, SPH), lambda i: (i, 0)),
            pl.BlockSpec((NB, HIDP), lambda i: (0, 0)),
            pl.BlockSpec((HIDP,), lambda i: (0,)),
            pl.BlockSpec((96, 352), lambda i: (0, 0)),
        ],
        out_specs=pl.BlockSpec((BE, HIDP), lambda i: (i, 0)),
        out_shape=jax.ShapeDtypeStruct((E, HIDP), jnp.float32),
    )(g_tab, rbf, fcut, rsh, Wr, br, sel)


def _combine_body(xs_ref, xsph_ref, parts_ref, ns_ref, nsph_ref):
    p = parts_ref[...]
    q = p[:, 0] + p[:, 1]
    ns_ref[...] = xs_ref[...] + q[0]
    sph = jnp.concatenate([q[1], q[2], q[3], q[4]], axis=1)[:, :SPH]
    nsph_ref[...] = xsph_ref[...] + sph


def _combine(x_scalar, x_spherical, parts):
    return pl.pallas_call(
        _combine_body,
        grid=(N // BN,),
        in_specs=[
            pl.BlockSpec((BN, NODE), lambda i: (i, 0)),
            pl.BlockSpec((BN, SPH), lambda i: (i, 0)),
            pl.BlockSpec((_NPASS, _NC, BN, 128), lambda i: (0, 0, i, 0)),
        ],
        out_specs=[
            pl.BlockSpec((BN, NODE), lambda i: (i, 0)),
            pl.BlockSpec((BN, SPH), lambda i: (i, 0)),
        ],
        out_shape=[
            jax.ShapeDtypeStruct((N, NODE), jnp.float32),
            jax.ShapeDtypeStruct((N, SPH), jnp.float32),
        ],
    )(x_scalar, x_spherical, parts)


_NC, _NS = 2, 16
_NW = _NC * _NS
_EPW = E // _NW
_GCH = 40

def _sc_mesh():
    return plsc.VectorSubcoreMesh(core_axis_name="c", subcore_axis_name="s")


_NGCH = _EPW // _GCH


def _gather_body(tab_hbm, dst_hbm, out_tab,
                 idx_all, buf_a, buf_b,
                 gsem_a, gsem_b, wsem_a, wsem_b):
    wid = lax.axis_index("s") * _NC + lax.axis_index("c")
    base = wid * _EPW
    pltpu.sync_copy(dst_hbm.at[pl.ds(base, _EPW)], idx_all)

    def gstart(chunk, buf, gsem):
        off = pl.multiple_of(chunk * _GCH, 8)
        pltpu.async_copy(tab_hbm.at[idx_all.at[pl.ds(off, _GCH)]], buf, gsem)

    def gwait(buf, gsem):
        pltpu.make_async_copy(tab_hbm.at[pl.ds(0, _GCH)], buf, gsem).wait()

    def wstart(chunk, buf, wsem):
        cb = pl.multiple_of(base + chunk * _GCH, 8)
        pltpu.async_copy(buf, out_tab.at[pl.ds(cb, _GCH)], wsem)

    def wwait(buf, wsem):
        pltpu.make_async_copy(buf, out_tab.at[pl.ds(0, _GCH)], wsem).wait()

    gstart(0, buf_a, gsem_a)
    gstart(1, buf_b, gsem_b)

    def pair(j, carry):
        gwait(buf_a, gsem_a)
        wstart(2 * j, buf_a, wsem_a)

        @pl.when(2 * j + 1 < _NGCH)
        def _():
            gwait(buf_b, gsem_b)
            wstart(2 * j + 1, buf_b, wsem_b)

        @pl.when(2 * j + 2 < _NGCH)
        def _():
            wwait(buf_a, wsem_a)
            gstart(2 * j + 2, buf_a, gsem_a)

        @pl.when(2 * j + 3 < _NGCH)
        def _():
            wwait(buf_b, wsem_b)
            gstart(2 * j + 3, buf_b, gsem_b)

        return carry

    lax.fori_loop(0, (_NGCH + 1) // 2, pair, 0)
    wwait(buf_a, wsem_a)
    wwait(buf_b, wsem_b)


def _sc_gather(table, dst):
    return pl.kernel(
        _gather_body,
        out_type=jax.ShapeDtypeStruct((E, TBW), jnp.float32),
        mesh=_sc_mesh(),
        scratch_types=[
            pltpu.VMEM((_EPW,), jnp.int32),
            pltpu.VMEM((_GCH, TBW), jnp.float32),
            pltpu.VMEM((_GCH, TBW), jnp.float32),
            pltpu.SemaphoreType.DMA,
            pltpu.SemaphoreType.DMA,
            pltpu.SemaphoreType.DMA,
            pltpu.SemaphoreType.DMA,
        ],
    )(table, dst)


_SCH = 128
_NFULL = _EPW // _SCH
_REM = _EPW - _NFULL * _SCH
_NPASS = HIDP // 128
_EPC = E // _NC


def _scatter_body(msg_hbm, src_hbm, zeros_hbm, out_hbm,
                  idx_a, idx_b, idx_r, msg_a, msg_b, msg_r,
                  acc_sh, sem_a, sem_b):
    c = lax.axis_index("c")
    s = lax.axis_index("s")
    ebase = c * _EPC + s * _EPW

    def start(chunk, idx_v, msg_v, sem, col):
        eb = pl.multiple_of(ebase + chunk * _SCH, 8)
        pltpu.async_copy(src_hbm.at[pl.ds(eb, _SCH)], idx_v, sem)
        pltpu.async_copy(msg_hbm.at[pl.ds(eb, _SCH), pl.ds(col, 128)],
                         msg_v, sem)

    def wait_and_scatter(idx_v, msg_v, sem, col):
        pltpu.make_async_copy(src_hbm.at[pl.ds(0, _SCH)], idx_v, sem).wait()
        pltpu.make_async_copy(msg_hbm.at[pl.ds(0, _SCH), pl.ds(col, 128)],
                              msg_v, sem).wait()
        pltpu.sync_copy(msg_v, acc_sh.at[idx_v], add=True)

    def one_pass(p, carry):
        col = pl.multiple_of(p * 128, 128)
        @pl.when(s < _NS - 1)
        def _():
            pltpu.sync_copy(zeros_hbm.at[pl.ds(0, 624)],
                            acc_sh.at[pl.ds(s * 624, 624)])

        @pl.when(s == _NS - 1)
        def _():
            pltpu.sync_copy(zeros_hbm.at[pl.ds(0, 640)],
                            acc_sh.at[pl.ds(9360, 640)])

        plsc.subcore_barrier()

        start(0, idx_a, msg_a, sem_a, col)

        def pair(j, carry):
            @pl.when(2 * j + 1 < _NFULL)
            def _():
                start(2 * j + 1, idx_b, msg_b, sem_b, col)

            wait_and_scatter(idx_a, msg_a, sem_a, col)

            @pl.when(2 * j + 2 < _NFULL)
            def _():
                start(2 * j + 2, idx_a, msg_a, sem_a, col)

            @pl.when(2 * j + 1 < _NFULL)
            def _():
                wait_and_scatter(idx_b, msg_b, sem_b, col)

            return carry

        lax.fori_loop(0, (_NFULL + 1) // 2, pair, 0)

        rb = pl.multiple_of(ebase + _NFULL * _SCH, 8)
        pltpu.sync_copy(src_hbm.at[pl.ds(rb, _REM)], idx_r)
        pltpu.sync_copy(msg_hbm.at[pl.ds(rb, _REM), pl.ds(col, 128)], msg_r)
        pltpu.sync_copy(msg_r, acc_sh.at[idx_r], add=True)
        plsc.subcore_barrier()

        @pl.when(s < _NS - 1)
        def _():
            pltpu.sync_copy(acc_sh.at[pl.ds(s * 624, 624)],
                            out_hbm.at[p, c].at[pl.ds(s * 624, 624)])

        @pl.when(s == _NS - 1)
        def _():
            pltpu.sync_copy(acc_sh.at[pl.ds(9360, 640)],
                            out_hbm.at[p, c].at[pl.ds(9360, 640)])

        plsc.subcore_barrier()
        return carry

    lax.fori_loop(0, _NPASS, one_pass, 0)


def _sc_scatter(msg, src, zeros):
    return pl.kernel(
        _scatter_body,
        out_type=jax.ShapeDtypeStruct((_NPASS, _NC, N, 128), jnp.float32),
        mesh=_sc_mesh(),
        scratch_types=[
            pltpu.VMEM((_SCH,), jnp.int32),
            pltpu.VMEM((_SCH,), jnp.int32),
            pltpu.VMEM((_REM,), jnp.int32),
            pltpu.VMEM((_SCH, 128), jnp.float32),
            pltpu.VMEM((_SCH, 128), jnp.float32),
            pltpu.VMEM((_REM, 128), jnp.float32),
            pltpu.VMEM_SHARED((N, 128), jnp.float32),
            pltpu.SemaphoreType.DMA,
            pltpu.SemaphoreType.DMA,
        ],
    )(msg, src, zeros)


def kernel(x_scalar, x_spherical, rbf, fcut, rsh, edge_index, W1, b1, W2, b2, Wr, br):
    sel = jnp.asarray(_SEL_TAIL)
    W2p = jnp.pad(W2, ((0, 0), (0, HIDP - HID)))
    b2p = jnp.pad(b2, (0, HIDP - HID))
    Wrp = jnp.pad(Wr, ((0, 0), (0, HIDP - HID)))
    brp = jnp.pad(br, (0, HIDP - HID))
    table = _mlp(x_scalar, x_spherical, W1, b1, W2p, b2p)
    dst = edge_index[1]
    src = edge_index[0]
    g_tab = _sc_gather(table, dst)
    msg = _edge_math(g_tab, rbf, fcut, rsh, Wrp, brp, sel)
    zeros = jnp.zeros((640, 128), jnp.float32)
    parts = _sc_scatter(msg, src, zeros)
    return tuple(_combine(x_scalar, x_spherical, parts))


# speedup vs baseline: 3.2155x; 1.1569x over previous
import functools

import jax
import jax.numpy as jnp
import numpy as np
from jax import lax
from jax.experimental import pallas as pl
from jax.experimental.pallas import tpu as pltpu
from jax.experimental.pallas import tpu_sc as plsc

N = 10000
E = 160000
NODE = 128
NUM_IRR = 224
SPH = 480
NB = 20
HID = NODE + NUM_IRR * 2

_reps = np.array([1] * 128 + [3] * 64 + [5] * 32)
_col_of = np.repeat(np.arange(NUM_IRR), _reps)
_SEL_TAIL = np.zeros((96, 352), dtype=np.float32)
for _j, _c in enumerate(_col_of[128:]):
    _SEL_TAIL[_c - 128, _j] = 1.0

BN = 1000
BE = 640
HIDP = 640
SPHP = 512


def _mlp_body(x_ref, xsph_ref, w1_ref, b1_ref, w2_ref, b2_ref, o_ref):
    x = x_ref[...]
    h = jnp.dot(x, w1_ref[...], preferred_element_type=jnp.float32,
                precision=jax.lax.Precision.HIGHEST) + b1_ref[...]
    h = h * jax.nn.sigmoid(h)
    so = jnp.dot(h, w2_ref[...], preferred_element_type=jnp.float32,
                 precision=jax.lax.Precision.HIGHEST) + b2_ref[...]
    pad = jnp.zeros((so.shape[0], HIDP - SPH), jnp.float32)
    hi = jnp.concatenate([xsph_ref[...], pad], axis=1)

    def rnd(u):
        return (u + 0x7FFF + ((u >> 16) & 1)) >> 16

    ulo = rnd(jax.lax.bitcast_convert_type(so, jnp.uint32))
    uhi = rnd(jax.lax.bitcast_convert_type(hi, jnp.uint32))
    o_ref[...] = jax.lax.bitcast_convert_type(ulo | (uhi << 16), jnp.float32)


def _edge_body(gtab_ref, rbf_ref, fcut_ref, rsh_ref,
               wr_ref, br_ref, sel_ref, msg_ref):
    fw = jnp.dot(rbf_ref[...], wr_ref[...], preferred_element_type=jnp.float32,
                 precision=jax.lax.Precision.HIGHEST) + br_ref[...]
    fw = fw * fcut_ref[...]
    u = jax.lax.bitcast_convert_type(gtab_ref[...], jnp.uint32)
    gso = jax.lax.bitcast_convert_type(u << 16, jnp.float32)
    gsph = jax.lax.bitcast_convert_type(u & jnp.uint32(0xFFFF0000), jnp.float32)
    fo = gso * fw
    gs = fo[:, :NUM_IRR]
    ge = fo[:, NUM_IRR:2 * NUM_IRR]
    ms = fo[:, 2 * NUM_IRR:HID]
    sel = sel_ref[...]

    def rep(g):
        tail = jnp.dot(g[:, 128:].astype(jnp.bfloat16), sel.astype(jnp.bfloat16),
                       preferred_element_type=jnp.float32)
        return jnp.concatenate([g[:, :128], tail], axis=1)

    msph = gsph[:, :SPH] * rep(gs) + rsh_ref[...] * rep(ge)
    pad = jnp.zeros((msph.shape[0], HIDP - NODE - SPH), jnp.float32)
    msg_ref[...] = jnp.concatenate([ms, msph, pad], axis=1)


TBW = HIDP


def _mlp(x_scalar, x_spherical, W1, b1, W2, b2):
    return pl.pallas_call(
        _mlp_body,
        grid=(N // BN,),
        in_specs=[
            pl.BlockSpec((BN, NODE), lambda i: (i, 0)),
            pl.BlockSpec((BN, SPH), lambda i: (i, 0)),
            pl.BlockSpec((NODE, NODE), lambda i: (0, 0)),
            pl.BlockSpec((NODE,), lambda i: (0,)),
            pl.BlockSpec((NODE, HIDP), lambda i: (0, 0)),
            pl.BlockSpec((HIDP,), lambda i: (0,)),
        ],
        out_specs=pl.BlockSpec((BN, TBW), lambda i: (i, 0)),
        out_shape=jax.ShapeDtypeStruct((N, TBW), jnp.float32),
    )(x_scalar, x_spherical, W1, b1, W2, b2)


def _edge_math(g_tab, rbf, fcut, rsh, Wr, br, sel):
    return pl.pallas_call(
        _edge_body,
        grid=(E // BE,),
        in_specs=[
            pl.BlockSpec((BE, TBW), lambda i: (i, 0)),
            pl.BlockSpec((BE, NB), lambda i: (i, 0)),
            pl.BlockSpec((BE, 1), lambda i: (i, 0)),
            pl.BlockSpec((BE, SPH), lambda i: (i, 0)),
            pl.BlockSpec((NB, HIDP), lambda i: (0, 0)),
            pl.BlockSpec((HIDP,), lambda i: (0,)),
            pl.BlockSpec((96, 352), lambda i: (0, 0)),
        ],
        out_specs=pl.BlockSpec((BE, HIDP), lambda i: (i, 0)),
        out_shape=jax.ShapeDtypeStruct((E, HIDP), jnp.float32),
    )(g_tab, rbf, fcut, rsh, Wr, br, sel)


def _combine_body(xs_ref, xsph_ref, parts_ref, ns_ref, nsph_ref):
    p = parts_ref[...]
    q = p[:, 0] + p[:, 1]
    ns_ref[...] = xs_ref[...] + q[0]
    sph = jnp.concatenate([q[1], q[2], q[3], q[4]], axis=1)[:, :SPH]
    nsph_ref[...] = xsph_ref[...] + sph


def _combine(x_scalar, x_spherical, parts):
    return pl.pallas_call(
        _combine_body,
        grid=(N // BN,),
        in_specs=[
            pl.BlockSpec((BN, NODE), lambda i: (i, 0)),
            pl.BlockSpec((BN, SPH), lambda i: (i, 0)),
            pl.BlockSpec((_NPASS, _NC, BN, 128), lambda i: (0, 0, i, 0)),
        ],
        out_specs=[
            pl.BlockSpec((BN, NODE), lambda i: (i, 0)),
            pl.BlockSpec((BN, SPH), lambda i: (i, 0)),
        ],
        out_shape=[
            jax.ShapeDtypeStruct((N, NODE), jnp.float32),
            jax.ShapeDtypeStruct((N, SPH), jnp.float32),
        ],
    )(x_scalar, x_spherical, parts)


_NC, _NS = 2, 16
_NW = _NC * _NS
_EPW = E // _NW
_GCH = 40

def _sc_mesh():
    return plsc.VectorSubcoreMesh(core_axis_name="c", subcore_axis_name="s")


_NGCH = _EPW // _GCH


def _gather_body(tab_hbm, dst_hbm, out_tab,
                 idx_all, buf_a, buf_b,
                 gsem_a, gsem_b, wsem_a, wsem_b):
    wid = lax.axis_index("s") * _NC + lax.axis_index("c")
    base = wid * _EPW
    pltpu.sync_copy(dst_hbm.at[pl.ds(base, _EPW)], idx_all)

    def gstart(chunk, buf, gsem):
        off = pl.multiple_of(chunk * _GCH, 8)
        pltpu.async_copy(tab_hbm.at[idx_all.at[pl.ds(off, _GCH)]], buf, gsem)

    def gwait(buf, gsem):
        pltpu.make_async_copy(tab_hbm.at[pl.ds(0, _GCH)], buf, gsem).wait()

    def wstart(chunk, buf, wsem):
        cb = pl.multiple_of(base + chunk * _GCH, 8)
        pltpu.async_copy(buf, out_tab.at[pl.ds(cb, _GCH)], wsem)

    def wwait(buf, wsem):
        pltpu.make_async_copy(buf, out_tab.at[pl.ds(0, _GCH)], wsem).wait()

    gstart(0, buf_a, gsem_a)
    gstart(1, buf_b, gsem_b)

    def pair(j, carry):
        gwait(buf_a, gsem_a)
        wstart(2 * j, buf_a, wsem_a)

        @pl.when(2 * j + 1 < _NGCH)
        def _():
            gwait(buf_b, gsem_b)
            wstart(2 * j + 1, buf_b, wsem_b)

        @pl.when(2 * j + 2 < _NGCH)
        def _():
            wwait(buf_a, wsem_a)
            gstart(2 * j + 2, buf_a, gsem_a)

        @pl.when(2 * j + 3 < _NGCH)
        def _():
            wwait(buf_b, wsem_b)
            gstart(2 * j + 3, buf_b, gsem_b)

        return carry

    lax.fori_loop(0, (_NGCH + 1) // 2, pair, 0)
    wwait(buf_a, wsem_a)
    wwait(buf_b, wsem_b)


def _sc_gather(table, dst):
    return pl.kernel(
        _gather_body,
        out_type=jax.ShapeDtypeStruct((E, TBW), jnp.float32),
        mesh=_sc_mesh(),
        scratch_types=[
            pltpu.VMEM((_EPW,), jnp.int32),
            pltpu.VMEM((_GCH, TBW), jnp.float32),
            pltpu.VMEM((_GCH, TBW), jnp.float32),
            pltpu.SemaphoreType.DMA,
            pltpu.SemaphoreType.DMA,
            pltpu.SemaphoreType.DMA,
            pltpu.SemaphoreType.DMA,
        ],
    )(table, dst)


_SCH = 128
_NFULL = _EPW // _SCH
_REM = _EPW - _NFULL * _SCH
_NPASS = HIDP // 128
_EPC = E // _NC


def _scatter_body(msg_hbm, src_hbm, zeros_hbm, out_hbm,
                  idx_a, idx_b, idx_r, msg_a, msg_b, msg_r,
                  acc_sh, sem_a, sem_b):
    c = lax.axis_index("c")
    s = lax.axis_index("s")
    ebase = c * _EPC + s * _EPW

    def start(chunk, idx_v, msg_v, sem, col):
        eb = pl.multiple_of(ebase + chunk * _SCH, 8)
        pltpu.async_copy(src_hbm.at[pl.ds(eb, _SCH)], idx_v, sem)
        pltpu.async_copy(msg_hbm.at[pl.ds(eb, _SCH), pl.ds(col, 128)],
                         msg_v, sem)

    def wait_and_scatter(idx_v, msg_v, sem, col):
        pltpu.make_async_copy(src_hbm.at[pl.ds(0, _SCH)], idx_v, sem).wait()
        pltpu.make_async_copy(msg_hbm.at[pl.ds(0, _SCH), pl.ds(col, 128)],
                              msg_v, sem).wait()
        pltpu.sync_copy(msg_v, acc_sh.at[idx_v], add=True)

    def one_pass(p, carry):
        col = pl.multiple_of(p * 128, 128)
        @pl.when(s < _NS - 1)
        def _():
            pltpu.sync_copy(zeros_hbm.at[pl.ds(0, 624)],
                            acc_sh.at[pl.ds(s * 624, 624)])

        @pl.when(s == _NS - 1)
        def _():
            pltpu.sync_copy(zeros_hbm.at[pl.ds(0, 640)],
                            acc_sh.at[pl.ds(9360, 640)])

        plsc.subcore_barrier()

        start(0, idx_a, msg_a, sem_a, col)

        def pair(j, carry):
            @pl.when(2 * j + 1 < _NFULL)
            def _():
                start(2 * j + 1, idx_b, msg_b, sem_b, col)

            wait_and_scatter(idx_a, msg_a, sem_a, col)

            @pl.when(2 * j + 2 < _NFULL)
            def _():
                start(2 * j + 2, idx_a, msg_a, sem_a, col)

            @pl.when(2 * j + 1 < _NFULL)
            def _():
                wait_and_scatter(idx_b, msg_b, sem_b, col)

            return carry

        lax.fori_loop(0, (_NFULL + 1) // 2, pair, 0)

        rb = pl.multiple_of(ebase + _NFULL * _SCH, 8)
        pltpu.sync_copy(src_hbm.at[pl.ds(rb, _REM)], idx_r)
        pltpu.sync_copy(msg_hbm.at[pl.ds(rb, _REM), pl.ds(col, 128)], msg_r)
        pltpu.sync_copy(msg_r, acc_sh.at[idx_r], add=True)
        plsc.subcore_barrier()

        @pl.when(s < _NS - 1)
        def _():
            pltpu.sync_copy(acc_sh.at[pl.ds(s * 624, 624)],
                            out_hbm.at[p, c].at[pl.ds(s * 624, 624)])

        @pl.when(s == _NS - 1)
        def _():
            pltpu.sync_copy(acc_sh.at[pl.ds(9360, 640)],
                            out_hbm.at[p, c].at[pl.ds(9360, 640)])

        plsc.subcore_barrier()
        return carry

    lax.fori_loop(0, _NPASS, one_pass, 0)


def _sc_scatter(msg, src, zeros):
    return pl.kernel(
        _scatter_body,
        out_type=jax.ShapeDtypeStruct((_NPASS, _NC, N, 128), jnp.float32),
        mesh=_sc_mesh(),
        scratch_types=[
            pltpu.VMEM((_SCH,), jnp.int32),
            pltpu.VMEM((_SCH,), jnp.int32),
            pltpu.VMEM((_REM,), jnp.int32),
            pltpu.VMEM((_SCH, 128), jnp.float32),
            pltpu.VMEM((_SCH, 128), jnp.float32),
            pltpu.VMEM((_REM, 128), jnp.float32),
            pltpu.VMEM_SHARED((N, 128), jnp.float32),
            pltpu.SemaphoreType.DMA,
            pltpu.SemaphoreType.DMA,
        ],
    )(msg, src, zeros)


def kernel(x_scalar, x_spherical, rbf, fcut, rsh, edge_index, W1, b1, W2, b2, Wr, br):
    sel = jnp.asarray(_SEL_TAIL)
    W2p = jnp.pad(W2, ((0, 0), (0, HIDP - HID)))
    b2p = jnp.pad(b2, (0, HIDP - HID))
    Wrp = jnp.pad(Wr, ((0, 0), (0, HIDP - HID)))
    brp = jnp.pad(br, (0, HIDP - HID))
    table = _mlp(x_scalar, x_spherical, W1, b1, W2p, b2p)
    dst = edge_index[1]
    src = edge_index[0]
    g_tab = _sc_gather(table, dst)
    msg = _edge_math(g_tab, rbf, fcut, rsh, Wrp, brp, sel)
    zeros = jnp.zeros((640, 128), jnp.float32)
    parts = _sc_scatter(msg, src, zeros)
    return tuple(_combine(x_scalar, x_spherical, parts))


# speedup vs baseline: 3.5813x; 1.1138x over previous
import functools

import jax
import jax.numpy as jnp
import numpy as np
from jax import lax
from jax.experimental import pallas as pl
from jax.experimental.pallas import tpu as pltpu
from jax.experimental.pallas import tpu_sc as plsc

N = 10000
E = 160000
NODE = 128
NUM_IRR = 224
SPH = 480
NB = 20
HID = NODE + NUM_IRR * 2

_reps = np.array([1] * 128 + [3] * 64 + [5] * 32)
_col_of = np.repeat(np.arange(NUM_IRR), _reps)
_SEL_TAIL = np.zeros((96, 352), dtype=np.float32)
for _j, _c in enumerate(_col_of[128:]):
    _SEL_TAIL[_c - 128, _j] = 1.0
_SEL2 = np.zeros((192, 704), dtype=np.float32)
_SEL2[:96, :352] = _SEL_TAIL
_SEL2[96:, 352:] = _SEL_TAIL

BN = 1000
BE = 1280
HIDP = 640
SPHP = 512


def _mlp_body(x_ref, xsph_ref, w1_ref, b1_ref, w2_ref, b2_ref, o_ref):
    x = x_ref[...]
    h = jnp.dot(x, w1_ref[...], preferred_element_type=jnp.float32,
                precision=jax.lax.Precision.HIGHEST) + b1_ref[...]
    h = h * jax.nn.sigmoid(h)
    so = jnp.dot(h, w2_ref[...], preferred_element_type=jnp.float32,
                 precision=jax.lax.Precision.HIGHEST) + b2_ref[...]
    pad = jnp.zeros((so.shape[0], HIDP - SPH), jnp.float32)
    hi = jnp.concatenate([xsph_ref[...], pad], axis=1)

    def rnd(u):
        return (u + 0x7FFF + ((u >> 16) & 1)) >> 16

    ulo = rnd(jax.lax.bitcast_convert_type(so, jnp.uint32))
    uhi = rnd(jax.lax.bitcast_convert_type(hi, jnp.uint32))
    o_ref[...] = jax.lax.bitcast_convert_type(ulo | (uhi << 16), jnp.float32)


def _edge_body(gtab_ref, rbf_ref, fcut_ref, rsh_ref,
               wr_ref, br_ref, sel_ref, msg_ref):
    fw = jnp.dot(rbf_ref[...].astype(jnp.bfloat16), wr_ref[...],
                 preferred_element_type=jnp.float32) + br_ref[...]
    fw = fw * fcut_ref[...]
    u = jax.lax.bitcast_convert_type(gtab_ref[...], jnp.uint32)
    gso = jax.lax.bitcast_convert_type(u << 16, jnp.float32)
    gsph = jax.lax.bitcast_convert_type(u & jnp.uint32(0xFFFF0000), jnp.float32)
    fo = gso * fw
    ms = fo[:, 2 * NUM_IRR:HID]
    gtails = jnp.concatenate([fo[:, 128:NUM_IRR], fo[:, NUM_IRR + 128:2 * NUM_IRR]],
                             axis=1).astype(jnp.bfloat16)
    tails = jnp.dot(gtails, sel_ref[...], preferred_element_type=jnp.float32)
    rep_gs = jnp.concatenate([fo[:, :128], tails[:, :352]], axis=1)
    rep_ge = jnp.concatenate([fo[:, NUM_IRR:NUM_IRR + 128], tails[:, 352:]],
                             axis=1)
    msph = gsph[:, :SPH] * rep_gs + rsh_ref[...] * rep_ge
    pad = jnp.zeros((msph.shape[0], HIDP - NODE - SPH), jnp.float32)
    msg_ref[...] = jnp.concatenate([ms, msph, pad], axis=1)


TBW = HIDP


def _mlp(x_scalar, x_spherical, W1, b1, W2, b2):
    return pl.pallas_call(
        _mlp_body,
        grid=(N // BN,),
        in_specs=[
            pl.BlockSpec((BN, NODE), lambda i: (i, 0)),
            pl.BlockSpec((BN, SPH), lambda i: (i, 0)),
            pl.BlockSpec((NODE, NODE), lambda i: (0, 0)),
            pl.BlockSpec((NODE,), lambda i: (0,)),
            pl.BlockSpec((NODE, HIDP), lambda i: (0, 0)),
            pl.BlockSpec((HIDP,), lambda i: (0,)),
        ],
        out_specs=pl.BlockSpec((BN, TBW), lambda i: (i, 0)),
        out_shape=jax.ShapeDtypeStruct((N, TBW), jnp.float32),
    )(x_scalar, x_spherical, W1, b1, W2, b2)


def _edge_math(g_tab, rbf, fcut, rsh, Wr, br, sel):
    return pl.pallas_call(
        _edge_body,
        grid=(E // BE,),
        in_specs=[
            pl.BlockSpec((BE, TBW), lambda i: (i, 0)),
            pl.BlockSpec((BE, NB), lambda i: (i, 0)),
            pl.BlockSpec((BE, 1), lambda i: (i, 0)),
            pl.BlockSpec((BE, SPH), lambda i: (i, 0)),
            pl.BlockSpec((NB, HIDP), lambda i: (0, 0)),
            pl.BlockSpec((HIDP,), lambda i: (0,)),
            pl.BlockSpec((192, 704), lambda i: (0, 0)),
        ],
        out_specs=pl.BlockSpec((BE, HIDP), lambda i: (i, 0)),
        out_shape=jax.ShapeDtypeStruct((E, HIDP), jnp.float32),
    )(g_tab, rbf, fcut, rsh, Wr, br, sel)


def _combine_body(xs_ref, xsph_ref, parts_ref, ns_ref, nsph_ref):
    p = parts_ref[...]
    q = p[:, 0] + p[:, 1]
    ns_ref[...] = xs_ref[...] + q[0]
    sph = jnp.concatenate([q[1], q[2], q[3], q[4]], axis=1)[:, :SPH]
    nsph_ref[...] = xsph_ref[...] + sph


def _combine(x_scalar, x_spherical, parts):
    return pl.pallas_call(
        _combine_body,
        grid=(N // BN,),
        in_specs=[
            pl.BlockSpec((BN, NODE), lambda i: (i, 0)),
            pl.BlockSpec((BN, SPH), lambda i: (i, 0)),
            pl.BlockSpec((_NPASS, _NC, BN, 128), lambda i: (0, 0, i, 0)),
        ],
        out_specs=[
            pl.BlockSpec((BN, NODE), lambda i: (i, 0)),
            pl.BlockSpec((BN, SPH), lambda i: (i, 0)),
        ],
        out_shape=[
            jax.ShapeDtypeStruct((N, NODE), jnp.float32),
            jax.ShapeDtypeStruct((N, SPH), jnp.float32),
        ],
    )(x_scalar, x_spherical, parts)


_NC, _NS = 2, 16
_NW = _NC * _NS
_EPW = E // _NW
_GCH = 40

def _sc_mesh():
    return plsc.VectorSubcoreMesh(core_axis_name="c", subcore_axis_name="s")


_NGCH = _EPW // _GCH


def _gather_body(tab_hbm, dst_hbm, out_tab,
                 idx_all, buf_a, buf_b,
                 gsem_a, gsem_b, wsem_a, wsem_b):
    wid = lax.axis_index("s") * _NC + lax.axis_index("c")
    base = wid * _EPW
    pltpu.sync_copy(dst_hbm.at[pl.ds(base, _EPW)], idx_all)

    def gstart(chunk, buf, gsem):
        off = pl.multiple_of(chunk * _GCH, 8)
        pltpu.async_copy(tab_hbm.at[idx_all.at[pl.ds(off, _GCH)]], buf, gsem)

    def gwait(buf, gsem):
        pltpu.make_async_copy(tab_hbm.at[pl.ds(0, _GCH)], buf, gsem).wait()

    def wstart(chunk, buf, wsem):
        cb = pl.multiple_of(base + chunk * _GCH, 8)
        pltpu.async_copy(buf, out_tab.at[pl.ds(cb, _GCH)], wsem)

    def wwait(buf, wsem):
        pltpu.make_async_copy(buf, out_tab.at[pl.ds(0, _GCH)], wsem).wait()

    gstart(0, buf_a, gsem_a)
    gstart(1, buf_b, gsem_b)

    def pair(j, carry):
        gwait(buf_a, gsem_a)
        wstart(2 * j, buf_a, wsem_a)

        @pl.when(2 * j + 1 < _NGCH)
        def _():
            gwait(buf_b, gsem_b)
            wstart(2 * j + 1, buf_b, wsem_b)

        @pl.when(2 * j + 2 < _NGCH)
        def _():
            wwait(buf_a, wsem_a)
            gstart(2 * j + 2, buf_a, gsem_a)

        @pl.when(2 * j + 3 < _NGCH)
        def _():
            wwait(buf_b, wsem_b)
            gstart(2 * j + 3, buf_b, gsem_b)

        return carry

    lax.fori_loop(0, (_NGCH + 1) // 2, pair, 0)
    wwait(buf_a, wsem_a)
    wwait(buf_b, wsem_b)


def _sc_gather(table, dst):
    return pl.kernel(
        _gather_body,
        out_type=jax.ShapeDtypeStruct((E, TBW), jnp.float32),
        mesh=_sc_mesh(),
        scratch_types=[
            pltpu.VMEM((_EPW,), jnp.int32),
            pltpu.VMEM((_GCH, TBW), jnp.float32),
            pltpu.VMEM((_GCH, TBW), jnp.float32),
            pltpu.SemaphoreType.DMA,
            pltpu.SemaphoreType.DMA,
            pltpu.SemaphoreType.DMA,
            pltpu.SemaphoreType.DMA,
        ],
    )(table, dst)


_SCH = 128
_NFULL = _EPW // _SCH
_REM = _EPW - _NFULL * _SCH
_NPASS = HIDP // 128
_EPC = E // _NC


def _scatter_body(msg_hbm, src_hbm, zeros_hbm, out_hbm,
                  idx_a, idx_b, idx_r, msg_a, msg_b, msg_r,
                  acc_sh, sem_a, sem_b):
    c = lax.axis_index("c")
    s = lax.axis_index("s")
    ebase = c * _EPC + s * _EPW

    def start(chunk, idx_v, msg_v, sem, col):
        eb = pl.multiple_of(ebase + chunk * _SCH, 8)
        pltpu.async_copy(src_hbm.at[pl.ds(eb, _SCH)], idx_v, sem)
        pltpu.async_copy(msg_hbm.at[pl.ds(eb, _SCH), pl.ds(col, 128)],
                         msg_v, sem)

    def wait_and_scatter(idx_v, msg_v, sem, col):
        pltpu.make_async_copy(src_hbm.at[pl.ds(0, _SCH)], idx_v, sem).wait()
        pltpu.make_async_copy(msg_hbm.at[pl.ds(0, _SCH), pl.ds(col, 128)],
                              msg_v, sem).wait()
        pltpu.sync_copy(msg_v, acc_sh.at[idx_v], add=True)

    def one_pass(p, carry):
        col = pl.multiple_of(p * 128, 128)
        @pl.when(s < _NS - 1)
        def _():
            pltpu.sync_copy(zeros_hbm.at[pl.ds(0, 624)],
                            acc_sh.at[pl.ds(s * 624, 624)])

        @pl.when(s == _NS - 1)
        def _():
            pltpu.sync_copy(zeros_hbm.at[pl.ds(0, 640)],
                            acc_sh.at[pl.ds(9360, 640)])

        plsc.subcore_barrier()

        start(0, idx_a, msg_a, sem_a, col)

        def pair(j, carry):
            @pl.when(2 * j + 1 < _NFULL)
            def _():
                start(2 * j + 1, idx_b, msg_b, sem_b, col)

            wait_and_scatter(idx_a, msg_a, sem_a, col)

            @pl.when(2 * j + 2 < _NFULL)
            def _():
                start(2 * j + 2, idx_a, msg_a, sem_a, col)

            @pl.when(2 * j + 1 < _NFULL)
            def _():
                wait_and_scatter(idx_b, msg_b, sem_b, col)

            return carry

        lax.fori_loop(0, (_NFULL + 1) // 2, pair, 0)

        rb = pl.multiple_of(ebase + _NFULL * _SCH, 8)
        pltpu.sync_copy(src_hbm.at[pl.ds(rb, _REM)], idx_r)
        pltpu.sync_copy(msg_hbm.at[pl.ds(rb, _REM), pl.ds(col, 128)], msg_r)
        pltpu.sync_copy(msg_r, acc_sh.at[idx_r], add=True)
        plsc.subcore_barrier()

        @pl.when(s < _NS - 1)
        def _():
            pltpu.sync_copy(acc_sh.at[pl.ds(s * 624, 624)],
                            out_hbm.at[p, c].at[pl.ds(s * 624, 624)])

        @pl.when(s == _NS - 1)
        def _():
            pltpu.sync_copy(acc_sh.at[pl.ds(9360, 640)],
                            out_hbm.at[p, c].at[pl.ds(9360, 640)])

        plsc.subcore_barrier()
        return carry

    lax.fori_loop(0, _NPASS, one_pass, 0)


def _sc_scatter(msg, src, zeros):
    return pl.kernel(
        _scatter_body,
        out_type=jax.ShapeDtypeStruct((_NPASS, _NC, N, 128), jnp.float32),
        mesh=_sc_mesh(),
        scratch_types=[
            pltpu.VMEM((_SCH,), jnp.int32),
            pltpu.VMEM((_SCH,), jnp.int32),
            pltpu.VMEM((_REM,), jnp.int32),
            pltpu.VMEM((_SCH, 128), jnp.float32),
            pltpu.VMEM((_SCH, 128), jnp.float32),
            pltpu.VMEM((_REM, 128), jnp.float32),
            pltpu.VMEM_SHARED((N, 128), jnp.float32),
            pltpu.SemaphoreType.DMA,
            pltpu.SemaphoreType.DMA,
        ],
    )(msg, src, zeros)


def kernel(x_scalar, x_spherical, rbf, fcut, rsh, edge_index, W1, b1, W2, b2, Wr, br):
    W2p = jnp.pad(W2, ((0, 0), (0, HIDP - HID)))
    b2p = jnp.pad(b2, (0, HIDP - HID))
    Wrp = jnp.pad(Wr, ((0, 0), (0, HIDP - HID)))
    brp = jnp.pad(br, (0, HIDP - HID))
    table = _mlp(x_scalar, x_spherical, W1, b1, W2p, b2p)
    sel2 = jnp.asarray(_SEL2).astype(jnp.bfloat16)
    dst = edge_index[1]
    src = edge_index[0]
    g_tab = _sc_gather(table, dst)
    msg = _edge_math(g_tab, rbf, fcut, rsh, Wrp.astype(jnp.bfloat16), brp, sel2)
    zeros = jnp.zeros((640, 128), jnp.float32)
    parts = _sc_scatter(msg, src, zeros)
    return tuple(_combine(x_scalar, x_spherical, parts))
